# baseline (speedup 1.0000x reference)
import jax
import jax.numpy as jnp
from jax.experimental import pallas as pl

L_BLOCK = 25


def _splat_kernel(pe_ref, out_ref):
    out_ref[...] = jnp.broadcast_to(pe_ref[...], out_ref.shape)


def kernel(x, pe_weight):
    batch = x.shape[0]
    max_len, dim = pe_weight.shape
    pe3d = pe_weight.reshape(max_len, dim, 1)
    out_t = pl.pallas_call(
        _splat_kernel,
        grid=(max_len // L_BLOCK,),
        in_specs=[pl.BlockSpec((L_BLOCK, dim, 1), lambda i: (i, 0, 0))],
        out_specs=pl.BlockSpec((L_BLOCK, dim, batch), lambda i: (i, 0, 0)),
        out_shape=jax.ShapeDtypeStruct((max_len, dim, batch), pe_weight.dtype),
    )(pe3d)
    return out_t.transpose(2, 0, 1)


# speedup vs baseline: 1.0251x; 1.0251x over previous
import jax
import jax.numpy as jnp
from jax.experimental import pallas as pl

L_BLOCK = 10


def _splat_kernel(pe_ref, out_ref):
    out_ref[...] = jnp.broadcast_to(pe_ref[...], out_ref.shape)


def kernel(x, pe_weight):
    batch = x.shape[0]
    max_len, dim = pe_weight.shape
    pe3d = pe_weight.reshape(max_len, dim, 1)
    out_t = pl.pallas_call(
        _splat_kernel,
        grid=(max_len // L_BLOCK,),
        in_specs=[pl.BlockSpec((L_BLOCK, dim, 1), lambda i: (i, 0, 0))],
        out_specs=pl.BlockSpec((L_BLOCK, dim, batch), lambda i: (i, 0, 0)),
        out_shape=jax.ShapeDtypeStruct((max_len, dim, batch), pe_weight.dtype),
    )(pe3d)
    return out_t.transpose(2, 0, 1)


# speedup vs baseline: 1.0999x; 1.0729x over previous
import jax
import jax.numpy as jnp
from jax.experimental import pallas as pl

L_BLOCK = 8


def _splat_kernel(pe_ref, out_ref):
    pe = pe_ref[...]
    out_ref[...] = jnp.broadcast_to(pe[:, :, None], out_ref.shape)


def kernel(x, pe_weight):
    batch = x.shape[0]
    max_len, dim = pe_weight.shape
    out_t = pl.pallas_call(
        _splat_kernel,
        grid=(max_len // L_BLOCK,),
        in_specs=[pl.BlockSpec((L_BLOCK, dim), lambda i: (i, 0))],
        out_specs=pl.BlockSpec((L_BLOCK, dim, batch), lambda i: (i, 0, 0)),
        out_shape=jax.ShapeDtypeStruct((max_len, dim, batch), pe_weight.dtype),
    )(pe_weight)
    return out_t.transpose(2, 0, 1)


# speedup vs baseline: 1.1100x; 1.0092x over previous
import jax
import jax.numpy as jnp
from jax.experimental import pallas as pl

L_BLOCK = 8


def _splat_kernel(pe_ref, out_ref):
    i = pl.program_id(0)
    pe = pe_ref[pl.ds(i * L_BLOCK, L_BLOCK), :]
    out_ref[...] = jnp.broadcast_to(pe[:, :, None], out_ref.shape)


def kernel(x, pe_weight):
    batch = x.shape[0]
    max_len, dim = pe_weight.shape
    out_t = pl.pallas_call(
        _splat_kernel,
        grid=(max_len // L_BLOCK,),
        in_specs=[pl.BlockSpec((max_len, dim), lambda i: (0, 0))],
        out_specs=pl.BlockSpec((L_BLOCK, dim, batch), lambda i: (i, 0, 0)),
        out_shape=jax.ShapeDtypeStruct((max_len, dim, batch), pe_weight.dtype),
    )(pe_weight)
    return out_t.transpose(2, 0, 1)


# speedup vs baseline: 1.1211x; 1.0100x over previous
import jax
import jax.numpy as jnp
from jax.experimental import pallas as pl

L_BLOCK = 4


def _splat_kernel(pe_ref, out_ref):
    i = pl.program_id(0)
    pe = pe_ref[pl.ds(i * L_BLOCK, L_BLOCK), :]
    out_ref[...] = jnp.broadcast_to(pe[:, :, None], out_ref.shape)


def kernel(x, pe_weight):
    batch = x.shape[0]
    max_len, dim = pe_weight.shape
    out_t = pl.pallas_call(
        _splat_kernel,
        grid=(max_len // L_BLOCK,),
        in_specs=[pl.BlockSpec((max_len, dim), lambda i: (0, 0))],
        out_specs=pl.BlockSpec((L_BLOCK, dim, batch), lambda i: (i, 0, 0)),
        out_shape=jax.ShapeDtypeStruct((max_len, dim, batch), pe_weight.dtype),
    )(pe_weight)
    return out_t.transpose(2, 0, 1)
